# baseline (speedup 1.0000x reference)
import jax
import jax.numpy as jnp
from jax.experimental import pallas as pl
from jax.experimental.pallas import tpu as pltpu

_HI = jax.lax.Precision.HIGHEST
_BF = jnp.bfloat16
_F8 = jnp.float8_e4m3fn


def _lrelu(v):
    return jnp.where(v >= 0, v, 0.2 * v)


def _pa_body(nchunk, adj_ref, x_ref, w1_ref, w2_ref,
             z1_ref, c1b_ref, adjf8_ref, s1b_ref):
    n = s1b_ref.shape[0]
    ck = n // nchunk

    @pl.when(pl.program_id(0) == 0)
    def _():
        def chunk(c, carry):
            s1b_ref[pl.ds(c * ck, ck), :] = _lrelu(
                jnp.dot(x_ref[pl.ds(c * ck, ck), :], w1_ref[...],
                        precision=_HI,
                        preferred_element_type=jnp.float32)).astype(_BF)
            return carry
        jax.lax.fori_loop(0, nchunk, chunk, 0)

    a = adj_ref[...]
    adjf8_ref[...] = a.astype(_F8)
    z1 = jnp.dot(a.astype(_BF), s1b_ref[...],
                 preferred_element_type=jnp.float32)
    z1_ref[...] = z1
    s2 = _lrelu(jnp.dot(z1, w2_ref[...], precision=_HI,
                        preferred_element_type=jnp.float32))
    c1b_ref[...] = jnp.concatenate([z1, s2], axis=1).astype(_BF)


def _pb_body(h1, adjf8_ref, c1b_ref, w3_ref, az1_ref, z2_ref, c2b_ref):
    r = jnp.dot(adjf8_ref[...].astype(_BF), c1b_ref[...],
                preferred_element_type=jnp.float32)
    az1_ref[...] = r[:, :h1]
    z2 = r[:, h1:]
    z2_ref[...] = z2
    s3 = jnp.dot(z2, w3_ref[...], precision=_HI,
                 preferred_element_type=jnp.float32)
    c2b_ref[...] = jnp.concatenate([z2, s3], axis=1).astype(_BF)


def _pcd_body(h2, bm, adjf8_ref, c2b_ref,
              az2_ref, zi_ref, zadj_ref, az3_ref, zib_ref):
    p = pl.program_id(0)
    i = pl.program_id(1)
    adjb = adjf8_ref[...].astype(_BF)

    @pl.when(p == 0)
    def _():
        r = jnp.dot(adjb, c2b_ref[...], preferred_element_type=jnp.float32)
        az2_ref[...] = r[:, :h2]
        zi = r[:, h2:]
        zi_ref[...] = zi
        zib_ref[pl.ds(i * bm, bm), :] = zi.astype(_BF)

    @pl.when(p == 1)
    def _():
        zcb = zib_ref[...]
        zrb = zib_ref[pl.ds(i * bm, bm), :]
        g = jax.lax.dot_general(zrb, zcb, (((1,), (1,)), ((), ())),
                                preferred_element_type=jnp.float32)
        zadj_ref[...] = 0.5 * (jnp.tanh(0.5 * g) + 1.0)
        az3_ref[...] = jnp.dot(adjb, zcb, preferred_element_type=jnp.float32)


def kernel(x, adj, W1, W2, W3):
    n, d_in = x.shape
    h1 = W1.shape[1]
    h2 = W2.shape[1]
    nz = W3.shape[1]
    f32 = jnp.float32

    bm_a = n // 25
    z1, c1b, adjf8 = pl.pallas_call(
        lambda *refs: _pa_body(10, *refs),
        grid=(n // bm_a,),
        in_specs=[pl.BlockSpec((bm_a, n), lambda i: (i, 0)),
                  pl.BlockSpec((n, d_in), lambda i: (0, 0)),
                  pl.BlockSpec((d_in, h1), lambda i: (0, 0)),
                  pl.BlockSpec((h1, h2), lambda i: (0, 0))],
        out_specs=[pl.BlockSpec((bm_a, h1), lambda i: (i, 0)),
                   pl.BlockSpec((bm_a, h1 + h2), lambda i: (i, 0)),
                   pl.BlockSpec((bm_a, n), lambda i: (i, 0))],
        out_shape=[jax.ShapeDtypeStruct((n, h1), f32),
                   jax.ShapeDtypeStruct((n, h1 + h2), _BF),
                   jax.ShapeDtypeStruct((n, n), _F8)],
        scratch_shapes=[pltpu.VMEM((n, h1), _BF)],
    )(adj, x, W1, W2)

    bm_b = n // 10
    az1, z2, c2b = pl.pallas_call(
        lambda *refs: _pb_body(h1, *refs),
        grid=(n // bm_b,),
        in_specs=[pl.BlockSpec((bm_b, n), lambda i: (i, 0)),
                  pl.BlockSpec((n, h1 + h2), lambda i: (0, 0)),
                  pl.BlockSpec((h2, nz), lambda i: (0, 0))],
        out_specs=[pl.BlockSpec((bm_b, h1), lambda i: (i, 0)),
                   pl.BlockSpec((bm_b, h2), lambda i: (i, 0)),
                   pl.BlockSpec((bm_b, h2 + nz), lambda i: (i, 0))],
        out_shape=[jax.ShapeDtypeStruct((n, h1), f32),
                   jax.ShapeDtypeStruct((n, h2), f32),
                   jax.ShapeDtypeStruct((n, h2 + nz), _BF)],
    )(adjf8, c1b, W3)

    bm_d = n // 25
    ns = n // bm_d
    last = ns - 1
    az2, z_igae, z_adj, az3 = pl.pallas_call(
        lambda *refs: _pcd_body(h2, bm_d, *refs),
        grid=(2, ns),
        in_specs=[pl.BlockSpec((bm_d, n), lambda p, i: (i, 0)),
                  pl.BlockSpec((n, h2 + nz), lambda p, i: (0, 0))],
        out_specs=[
            pl.BlockSpec((bm_d, h2),
                         lambda p, i: (jnp.where(p == 0, i, last), 0)),
            pl.BlockSpec((bm_d, nz),
                         lambda p, i: (jnp.where(p == 0, i, last), 0)),
            pl.BlockSpec((bm_d, n),
                         lambda p, i: (jnp.where(p == 1, i, 0), 0)),
            pl.BlockSpec((bm_d, nz),
                         lambda p, i: (jnp.where(p == 1, i, 0), 0)),
        ],
        out_shape=[jax.ShapeDtypeStruct((n, h2), f32),
                   jax.ShapeDtypeStruct((n, nz), f32),
                   jax.ShapeDtypeStruct((n, n), f32),
                   jax.ShapeDtypeStruct((n, nz), f32)],
        scratch_shapes=[pltpu.VMEM((n, nz), _BF)],
    )(adjf8, c2b)

    return (z_igae, z_adj, az1, az2, az3, z1, z2, z_igae)


# speedup vs baseline: 1.0261x; 1.0261x over previous
import jax
import jax.numpy as jnp
from jax.experimental import pallas as pl
from jax.experimental.pallas import tpu as pltpu

_HI = jax.lax.Precision.HIGHEST
_BF = jnp.bfloat16
_F8 = jnp.float8_e4m3fn


def _lrelu(v):
    return jnp.where(v >= 0, v, 0.2 * v)


def _pa_body(nchunk, adj_ref, x_ref, w1_ref, w2_ref,
             z1_ref, c1b_ref, adjf8_ref, s1b_ref):
    n = s1b_ref.shape[0]
    ck = n // nchunk

    @pl.when(pl.program_id(0) == 0)
    def _():
        def chunk(c, carry):
            s1b_ref[pl.ds(c * ck, ck), :] = _lrelu(
                jnp.dot(x_ref[pl.ds(c * ck, ck), :], w1_ref[...],
                        precision=_HI,
                        preferred_element_type=jnp.float32)).astype(_BF)
            return carry
        jax.lax.fori_loop(0, nchunk, chunk, 0)

    a = adj_ref[...]
    adjf8_ref[...] = a.astype(_F8)
    adjb = a.astype(_BF)
    z1 = jnp.dot(adjb, s1b_ref[...], preferred_element_type=jnp.float32)
    z1_ref[...] = z1
    s2 = _lrelu(jnp.dot(z1, w2_ref[...], precision=_HI,
                        preferred_element_type=jnp.float32))
    c1b_ref[...] = jnp.concatenate([z1, s2], axis=1).astype(_BF)


def _pb_body(h1, adjf8_ref, c1b_ref, w3_ref, az1_ref, z2_ref, c2b_ref):
    r = jnp.dot(adjf8_ref[...].astype(_BF), c1b_ref[...],
                preferred_element_type=jnp.float32)
    az1_ref[...] = r[:, :h1]
    z2 = r[:, h1:]
    z2_ref[...] = z2
    s3 = jnp.dot(z2, w3_ref[...], precision=_HI,
                 preferred_element_type=jnp.float32)
    c2b_ref[...] = jnp.concatenate([z2, s3], axis=1).astype(_BF)


def _pc_body(h2, adjf8_ref, c2b_ref, az2_ref, zi_ref, zib_ref):
    r = jnp.dot(adjf8_ref[...].astype(_BF), c2b_ref[...],
                preferred_element_type=jnp.float32)
    az2_ref[...] = r[:, :h2]
    zi = r[:, h2:]
    zi_ref[...] = zi
    zib_ref[...] = zi.astype(_BF)


def _pd_body(adjf8_ref, zrb_ref, zcb_ref, zadj_ref, az3_ref):
    zcb = zcb_ref[...]
    g = jax.lax.dot_general(zrb_ref[...], zcb, (((1,), (1,)), ((), ())),
                            preferred_element_type=jnp.float32)
    zadj_ref[...] = 0.5 * (jnp.tanh(0.5 * g) + 1.0)
    az3_ref[...] = jnp.dot(adjf8_ref[...].astype(_BF), zcb,
                           preferred_element_type=jnp.float32)


def kernel(x, adj, W1, W2, W3):
    n, d_in = x.shape
    h1 = W1.shape[1]
    h2 = W2.shape[1]
    nz = W3.shape[1]
    f32 = jnp.float32

    bm_a = n // 25
    z1, c1b, adjf8 = pl.pallas_call(
        lambda *refs: _pa_body(10, *refs),
        grid=(n // bm_a,),
        in_specs=[pl.BlockSpec((bm_a, n), lambda i: (i, 0)),
                  pl.BlockSpec((n, d_in), lambda i: (0, 0)),
                  pl.BlockSpec((d_in, h1), lambda i: (0, 0)),
                  pl.BlockSpec((h1, h2), lambda i: (0, 0))],
        out_specs=[pl.BlockSpec((bm_a, h1), lambda i: (i, 0)),
                   pl.BlockSpec((bm_a, h1 + h2), lambda i: (i, 0)),
                   pl.BlockSpec((bm_a, n), lambda i: (i, 0))],
        out_shape=[jax.ShapeDtypeStruct((n, h1), f32),
                   jax.ShapeDtypeStruct((n, h1 + h2), _BF),
                   jax.ShapeDtypeStruct((n, n), _F8)],
        scratch_shapes=[pltpu.VMEM((n, h1), _BF)],
    )(adj, x, W1, W2)

    bm_b = n // 10
    az1, z2, c2b = pl.pallas_call(
        lambda *refs: _pb_body(h1, *refs),
        grid=(n // bm_b,),
        in_specs=[pl.BlockSpec((bm_b, n), lambda i: (i, 0)),
                  pl.BlockSpec((n, h1 + h2), lambda i: (0, 0)),
                  pl.BlockSpec((h2, nz), lambda i: (0, 0))],
        out_specs=[pl.BlockSpec((bm_b, h1), lambda i: (i, 0)),
                   pl.BlockSpec((bm_b, h2), lambda i: (i, 0)),
                   pl.BlockSpec((bm_b, h2 + nz), lambda i: (i, 0))],
        out_shape=[jax.ShapeDtypeStruct((n, h1), f32),
                   jax.ShapeDtypeStruct((n, h2), f32),
                   jax.ShapeDtypeStruct((n, h2 + nz), _BF)],
    )(adjf8, c1b, W3)

    az2, z_igae, zib = pl.pallas_call(
        lambda *refs: _pc_body(h2, *refs),
        grid=(n // bm_b,),
        in_specs=[pl.BlockSpec((bm_b, n), lambda i: (i, 0)),
                  pl.BlockSpec((n, h2 + nz), lambda i: (0, 0))],
        out_specs=[pl.BlockSpec((bm_b, h2), lambda i: (i, 0)),
                   pl.BlockSpec((bm_b, nz), lambda i: (i, 0)),
                   pl.BlockSpec((bm_b, nz), lambda i: (i, 0))],
        out_shape=[jax.ShapeDtypeStruct((n, h2), f32),
                   jax.ShapeDtypeStruct((n, nz), f32),
                   jax.ShapeDtypeStruct((n, nz), _BF)],
    )(adjf8, c2b)

    bm_d = n // 25
    z_adj, az3 = pl.pallas_call(
        _pd_body,
        grid=(n // bm_d,),
        in_specs=[pl.BlockSpec((bm_d, n), lambda i: (i, 0)),
                  pl.BlockSpec((bm_d, nz), lambda i: (i, 0)),
                  pl.BlockSpec((n, nz), lambda i: (0, 0))],
        out_specs=[pl.BlockSpec((bm_d, n), lambda i: (i, 0)),
                   pl.BlockSpec((bm_d, nz), lambda i: (i, 0))],
        out_shape=[jax.ShapeDtypeStruct((n, n), f32),
                   jax.ShapeDtypeStruct((n, nz), f32)],
    )(adjf8, zib, zib)

    return (z_igae, z_adj, az1, az2, az3, z1, z2, z_igae)


# speedup vs baseline: 1.0268x; 1.0006x over previous
import jax
import jax.numpy as jnp
from jax.experimental import pallas as pl

_HI = jax.lax.Precision.HIGHEST
_BF = jnp.bfloat16
_F8 = jnp.float8_e4m3fn


def _lrelu(v):
    return jnp.where(v >= 0, v, 0.2 * v)


def _s1_body(x_ref, w1_ref, s1b_ref):
    s1 = _lrelu(jnp.dot(x_ref[...], w1_ref[...], precision=_HI,
                        preferred_element_type=jnp.float32))
    s1b_ref[...] = s1.astype(_BF)


def _pa_body(adj_ref, s1b_ref, w2_ref, z1_ref, c1b_ref, adjf8_ref):
    a = adj_ref[...]
    adjf8_ref[...] = a.astype(_F8)
    adjb = a.astype(_BF)
    z1 = jnp.dot(adjb, s1b_ref[...], preferred_element_type=jnp.float32)
    z1_ref[...] = z1
    s2 = _lrelu(jnp.dot(z1, w2_ref[...], precision=_HI,
                        preferred_element_type=jnp.float32))
    c1b_ref[...] = jnp.concatenate([z1, s2], axis=1).astype(_BF)


def _pb_body(h1, adjf8_ref, c1b_ref, w3_ref, az1_ref, z2_ref, c2b_ref):
    r = jnp.dot(adjf8_ref[...].astype(_BF), c1b_ref[...],
                preferred_element_type=jnp.float32)
    az1_ref[...] = r[:, :h1]
    z2 = r[:, h1:]
    z2_ref[...] = z2
    s3 = jnp.dot(z2, w3_ref[...], precision=_HI,
                 preferred_element_type=jnp.float32)
    c2b_ref[...] = jnp.concatenate([z2, s3], axis=1).astype(_BF)


def _pc_body(h2, adjf8_ref, c2b_ref, az2_ref, zi_ref, zib_ref):
    r = jnp.dot(adjf8_ref[...].astype(_BF), c2b_ref[...],
                preferred_element_type=jnp.float32)
    az2_ref[...] = r[:, :h2]
    zi = r[:, h2:]
    zi_ref[...] = zi
    zib_ref[...] = zi.astype(_BF)


def _pd_body(adjf8_ref, zrb_ref, zcb_ref, zadj_ref, az3_ref):
    zcb = zcb_ref[...]
    g = jax.lax.dot_general(zrb_ref[...], zcb, (((1,), (1,)), ((), ())),
                            preferred_element_type=jnp.float32)
    zadj_ref[...] = 0.5 * (jnp.tanh(0.5 * g) + 1.0)
    az3_ref[...] = jnp.dot(adjf8_ref[...].astype(_BF), zcb,
                           preferred_element_type=jnp.float32)


def kernel(x, adj, W1, W2, W3):
    n, d_in = x.shape
    h1 = W1.shape[1]
    h2 = W2.shape[1]
    nz = W3.shape[1]
    f32 = jnp.float32

    bm = n // 5
    s1b = pl.pallas_call(
        _s1_body,
        grid=(n // bm,),
        in_specs=[pl.BlockSpec((bm, d_in), lambda i: (i, 0)),
                  pl.BlockSpec((d_in, h1), lambda i: (0, 0))],
        out_specs=pl.BlockSpec((bm, h1), lambda i: (i, 0)),
        out_shape=jax.ShapeDtypeStruct((n, h1), _BF),
    )(x, W1)

    bm_a = n // 25
    z1, c1b, adjf8 = pl.pallas_call(
        _pa_body,
        grid=(n // bm_a,),
        in_specs=[pl.BlockSpec((bm_a, n), lambda i: (i, 0)),
                  pl.BlockSpec((n, h1), lambda i: (0, 0)),
                  pl.BlockSpec((h1, h2), lambda i: (0, 0))],
        out_specs=[pl.BlockSpec((bm_a, h1), lambda i: (i, 0)),
                   pl.BlockSpec((bm_a, h1 + h2), lambda i: (i, 0)),
                   pl.BlockSpec((bm_a, n), lambda i: (i, 0))],
        out_shape=[jax.ShapeDtypeStruct((n, h1), f32),
                   jax.ShapeDtypeStruct((n, h1 + h2), _BF),
                   jax.ShapeDtypeStruct((n, n), _F8)],
    )(adj, s1b, W2)

    bm_b = n // 10
    az1, z2, c2b = pl.pallas_call(
        lambda *refs: _pb_body(h1, *refs),
        grid=(n // bm_b,),
        in_specs=[pl.BlockSpec((bm_b, n), lambda i: (i, 0)),
                  pl.BlockSpec((n, h1 + h2), lambda i: (0, 0)),
                  pl.BlockSpec((h2, nz), lambda i: (0, 0))],
        out_specs=[pl.BlockSpec((bm_b, h1), lambda i: (i, 0)),
                   pl.BlockSpec((bm_b, h2), lambda i: (i, 0)),
                   pl.BlockSpec((bm_b, h2 + nz), lambda i: (i, 0))],
        out_shape=[jax.ShapeDtypeStruct((n, h1), f32),
                   jax.ShapeDtypeStruct((n, h2), f32),
                   jax.ShapeDtypeStruct((n, h2 + nz), _BF)],
    )(adjf8, c1b, W3)

    az2, z_igae, zib = pl.pallas_call(
        lambda *refs: _pc_body(h2, *refs),
        grid=(n // bm_b,),
        in_specs=[pl.BlockSpec((bm_b, n), lambda i: (i, 0)),
                  pl.BlockSpec((n, h2 + nz), lambda i: (0, 0))],
        out_specs=[pl.BlockSpec((bm_b, h2), lambda i: (i, 0)),
                   pl.BlockSpec((bm_b, nz), lambda i: (i, 0)),
                   pl.BlockSpec((bm_b, nz), lambda i: (i, 0))],
        out_shape=[jax.ShapeDtypeStruct((n, h2), f32),
                   jax.ShapeDtypeStruct((n, nz), f32),
                   jax.ShapeDtypeStruct((n, nz), _BF)],
    )(adjf8, c2b)

    bm_d = n // 50
    z_adj, az3 = pl.pallas_call(
        _pd_body,
        grid=(n // bm_d,),
        in_specs=[pl.BlockSpec((bm_d, n), lambda i: (i, 0)),
                  pl.BlockSpec((bm_d, nz), lambda i: (i, 0)),
                  pl.BlockSpec((n, nz), lambda i: (0, 0))],
        out_specs=[pl.BlockSpec((bm_d, n), lambda i: (i, 0)),
                   pl.BlockSpec((bm_d, nz), lambda i: (i, 0))],
        out_shape=[jax.ShapeDtypeStruct((n, n), f32),
                   jax.ShapeDtypeStruct((n, nz), f32)],
    )(adjf8, zib, zib)

    return (z_igae, z_adj, az1, az2, az3, z1, z2, z_igae)


# speedup vs baseline: 1.0300x; 1.0031x over previous
import jax
import jax.numpy as jnp
from jax.experimental import pallas as pl

_HI = jax.lax.Precision.HIGHEST
_BF = jnp.bfloat16
_F8 = jnp.float8_e4m3fn


def _lrelu(v):
    return jnp.where(v >= 0, v, 0.2 * v)


def _s1_body(x_ref, w1_ref, s1b_ref):
    s1 = _lrelu(jnp.dot(x_ref[...], w1_ref[...], precision=_HI,
                        preferred_element_type=jnp.float32))
    s1b_ref[...] = s1.astype(_BF)


def _pa_body(adj_ref, s1b_ref, w2_ref, z1_ref, c1b_ref, adjf8_ref):
    a = adj_ref[...]
    adjf8_ref[...] = a.astype(_F8)
    adjb = a.astype(_BF)
    z1 = jnp.dot(adjb, s1b_ref[...], preferred_element_type=jnp.float32)
    z1_ref[...] = z1
    s2 = _lrelu(jnp.dot(z1, w2_ref[...], precision=_HI,
                        preferred_element_type=jnp.float32))
    c1b_ref[...] = jnp.concatenate([z1, s2], axis=1).astype(_BF)


def _pb_body(h1, adjf8_ref, c1b_ref, w3_ref, az1_ref, z2_ref, c2b_ref):
    r = jnp.dot(adjf8_ref[...].astype(_BF), c1b_ref[...],
                preferred_element_type=jnp.float32)
    az1_ref[...] = r[:, :h1]
    z2 = r[:, h1:]
    z2_ref[...] = z2
    s3 = jnp.dot(z2, w3_ref[...], precision=_HI,
                 preferred_element_type=jnp.float32)
    c2b_ref[...] = jnp.concatenate([z2, s3], axis=1).astype(_BF)


def _pc_body(h2, adjf8_ref, c2b_ref, az2_ref, zi_ref, zib_ref):
    r = jnp.dot(adjf8_ref[...].astype(_BF), c2b_ref[...],
                preferred_element_type=jnp.float32)
    az2_ref[...] = r[:, :h2]
    zi = r[:, h2:]
    zi_ref[...] = zi
    zib_ref[...] = zi.astype(_BF)


def _pd_body(adjf8_ref, zrb_ref, zcb_ref, zadj_ref, az3_ref):
    zcb = zcb_ref[...]
    g = jax.lax.dot_general(zrb_ref[...], zcb, (((1,), (1,)), ((), ())),
                            preferred_element_type=jnp.float32)
    zadj_ref[...] = 0.5 * (jnp.tanh(0.5 * g) + 1.0)
    az3_ref[...] = jnp.dot(adjf8_ref[...].astype(_BF), zcb,
                           preferred_element_type=jnp.float32)


def kernel(x, adj, W1, W2, W3):
    n, d_in = x.shape
    h1 = W1.shape[1]
    h2 = W2.shape[1]
    nz = W3.shape[1]
    f32 = jnp.float32

    bm = n // 5
    s1b = pl.pallas_call(
        _s1_body,
        grid=(n // bm,),
        in_specs=[pl.BlockSpec((bm, d_in), lambda i: (i, 0)),
                  pl.BlockSpec((d_in, h1), lambda i: (0, 0))],
        out_specs=pl.BlockSpec((bm, h1), lambda i: (i, 0)),
        out_shape=jax.ShapeDtypeStruct((n, h1), _BF),
    )(x, W1)

    bm_a = n // 25
    z1, c1b, adjf8 = pl.pallas_call(
        _pa_body,
        grid=(n // bm_a,),
        in_specs=[pl.BlockSpec((bm_a, n), lambda i: (i, 0)),
                  pl.BlockSpec((n, h1), lambda i: (0, 0)),
                  pl.BlockSpec((h1, h2), lambda i: (0, 0))],
        out_specs=[pl.BlockSpec((bm_a, h1), lambda i: (i, 0)),
                   pl.BlockSpec((bm_a, h1 + h2), lambda i: (i, 0)),
                   pl.BlockSpec((bm_a, n), lambda i: (i, 0))],
        out_shape=[jax.ShapeDtypeStruct((n, h1), f32),
                   jax.ShapeDtypeStruct((n, h1 + h2), _BF),
                   jax.ShapeDtypeStruct((n, n), _F8)],
    )(adj, s1b, W2)

    bm_b = n // 10
    az1, z2, c2b = pl.pallas_call(
        lambda *refs: _pb_body(h1, *refs),
        grid=(n // bm_b,),
        in_specs=[pl.BlockSpec((bm_b, n), lambda i: (i, 0)),
                  pl.BlockSpec((n, h1 + h2), lambda i: (0, 0)),
                  pl.BlockSpec((h2, nz), lambda i: (0, 0))],
        out_specs=[pl.BlockSpec((bm_b, h1), lambda i: (i, 0)),
                   pl.BlockSpec((bm_b, h2), lambda i: (i, 0)),
                   pl.BlockSpec((bm_b, h2 + nz), lambda i: (i, 0))],
        out_shape=[jax.ShapeDtypeStruct((n, h1), f32),
                   jax.ShapeDtypeStruct((n, h2), f32),
                   jax.ShapeDtypeStruct((n, h2 + nz), _BF)],
    )(adjf8, c1b, W3)

    az2, z_igae, zib = pl.pallas_call(
        lambda *refs: _pc_body(h2, *refs),
        grid=(n // bm_b,),
        in_specs=[pl.BlockSpec((bm_b, n), lambda i: (i, 0)),
                  pl.BlockSpec((n, h2 + nz), lambda i: (0, 0))],
        out_specs=[pl.BlockSpec((bm_b, h2), lambda i: (i, 0)),
                   pl.BlockSpec((bm_b, nz), lambda i: (i, 0)),
                   pl.BlockSpec((bm_b, nz), lambda i: (i, 0))],
        out_shape=[jax.ShapeDtypeStruct((n, h2), f32),
                   jax.ShapeDtypeStruct((n, nz), f32),
                   jax.ShapeDtypeStruct((n, nz), _BF)],
    )(adjf8, c2b)

    bm_d = n // 25
    z_adj, az3 = pl.pallas_call(
        _pd_body,
        grid=(n // bm_d,),
        in_specs=[pl.BlockSpec((bm_d, n), lambda i: (i, 0)),
                  pl.BlockSpec((bm_d, nz), lambda i: (i, 0)),
                  pl.BlockSpec((n, nz), lambda i: (0, 0))],
        out_specs=[pl.BlockSpec((bm_d, n), lambda i: (i, 0)),
                   pl.BlockSpec((bm_d, nz), lambda i: (i, 0))],
        out_shape=[jax.ShapeDtypeStruct((n, n), f32),
                   jax.ShapeDtypeStruct((n, nz), f32)],
    )(adjf8, zib, zib)

    return (z_igae, z_adj, az1, az2, az3, z1, z2, z_igae)
